# baseline (speedup 1.0000x reference)
import functools

import jax
import jax.numpy as jnp
from jax.experimental import pallas as pl
from jax.experimental.pallas import tpu as pltpu


def _bn_body(emb_ref, g_ref, b_ref, x32_ref, xta_ref, *, v, v_pad):
    emb = emb_ref[...]
    mu = jnp.mean(emb, axis=0, keepdims=True)
    var = jnp.mean((emb - mu) ** 2, axis=0, keepdims=True)
    x = (emb - mu) * jax.lax.rsqrt(var + 1e-5) * g_ref[...] + b_ref[...]
    if v_pad > v:
        x32_ref[...] = jnp.concatenate(
            [x, jnp.zeros((v_pad - v, x.shape[1]), jnp.float32)], axis=0)
    else:
        x32_ref[...] = x
    xta = jnp.concatenate(
        [jnp.swapaxes(x.astype(jnp.bfloat16), 0, 1),
         jnp.ones((8, v), jnp.bfloat16)], axis=0)
    if v_pad > v:
        xta = jnp.concatenate(
            [xta, jnp.zeros((xta.shape[0], v_pad - v), jnp.bfloat16)], axis=1)
    xta_ref[...] = xta


def _passA_body(h_ref, xta_ref, ew_ref, h8_ref, ewt_ref, de_ref, acc_ref,
                *, v, d, tvr, nvr):
    j = pl.program_id(1)
    h = h_ref[...]
    if v % tvr:
        def _mask(hh):
            rows = jax.lax.broadcasted_iota(jnp.int32, hh.shape, 0) + j * tvr
            return jnp.where(rows < v, hh, 0.0)
        h = jax.lax.cond(j == nvr - 1, _mask, lambda hh: hh, h)
    hb = h.astype(jnp.bfloat16)
    h8_ref[...] = h.astype(jnp.int8)
    contrib = jax.lax.dot_general(xta_ref[...], hb, (((1,), (0,)), ((), ())),
                                  preferred_element_type=jnp.float32)

    @pl.when(j == 0)
    def _init():
        acc_ref[...] = contrib

    @pl.when(j > 0)
    def _acc():
        acc_ref[...] += contrib

    @pl.when(j == nvr - 1)
    def _emit():
        de = acc_ref[d:d + 8, :]
        scale = ew_ref[0:1, :] / jnp.clip(de[0:1, :], 1.0, None)
        ewt_ref[...] = jnp.swapaxes(
            (acc_ref[0:d, :] * scale).astype(jnp.bfloat16), 0, 1)
        de_ref[...] = de


def _passB_body(h8_ref, x32_ref, ewt_ref, w_ref, bb_ref, de_ref,
                ones8_ref, e2tb_ref, acc_ref, *, nvb):
    i = pl.program_id(0)
    h8 = h8_ref[...]
    hb = h8.astype(jnp.bfloat16)
    dv32 = jax.lax.dot_general(h8, ones8_ref[...], (((1,), (0,)), ((), ())),
                               preferred_element_type=jnp.int32)
    dv = dv32[:, 0:1].astype(jnp.float32)
    m = jax.lax.dot_general(hb, ewt_ref[...], (((1,), (0,)), ((), ())),
                            preferred_element_type=jnp.float32)
    m = m / jnp.clip(dv, 1.0, None)
    r = jax.nn.relu(
        jax.lax.dot_general(m.astype(jnp.bfloat16),
                            w_ref[...].astype(jnp.bfloat16),
                            (((1,), (0,)), ((), ())),
                            preferred_element_type=jnp.float32) + bb_ref[...])
    xo16 = (r + x32_ref[...]).astype(jnp.bfloat16)
    xot = jnp.swapaxes(xo16, 0, 1)
    contrib = jax.lax.dot_general(xot, hb, (((1,), (0,)), ((), ())),
                                  preferred_element_type=jnp.float32)

    @pl.when(i == 0)
    def _init():
        acc_ref[...] = contrib

    @pl.when(i > 0)
    def _acc():
        acc_ref[...] += contrib

    @pl.when(i == nvb - 1)
    def _emit():
        invde = 1.0 / jnp.clip(de_ref[0:1, :], 1.0, None)
        e2tb_ref[...] = (acc_ref[...] * invde).astype(jnp.bfloat16)


def _combine_body(ed_ref, ep_ref, em_ref, out_ref):
    dp = (ed_ref[...].astype(jnp.float32)
          + ep_ref[...].astype(jnp.float32))
    mm = em_ref[...].astype(jnp.float32)
    out_ref[...] = jnp.concatenate(
        [jnp.swapaxes(dp, 0, 1), jnp.swapaxes(mm, 0, 1)], axis=1)


def _one_vocab(emb, g, b, W, bb, ew, H):
    v, d = emb.shape
    n_e = H.shape[1]
    if v >= 512:
        pad512 = -(-v // 512) * 512
        tvr = 512
        for cand in (1024, 768):
            if -(-v // cand) * cand == pad512:
                tvr = cand
                break
    else:
        tvr = 128
    nvr = -(-v // tvr)
    v_pad = nvr * tvr
    te = 2048 if n_e > 2048 else n_e
    nte = -(-n_e // te)

    x32p, xta = pl.pallas_call(
        functools.partial(_bn_body, v=v, v_pad=v_pad),
        out_shape=[jax.ShapeDtypeStruct((v_pad, d), jnp.float32),
                   jax.ShapeDtypeStruct((d + 8, v_pad), jnp.bfloat16)],
    )(emb, g, b)

    h8, ewt16, de8 = pl.pallas_call(
        functools.partial(_passA_body, v=v, d=d, tvr=tvr, nvr=nvr),
        grid=(nte, nvr),
        in_specs=[pl.BlockSpec((tvr, te), lambda i, j: (j, i)),
                  pl.BlockSpec((d + 8, tvr), lambda i, j: (0, j)),
                  pl.BlockSpec((1, te), lambda i, j: (0, i))],
        out_specs=[pl.BlockSpec((tvr, te), lambda i, j: (j, i)),
                   pl.BlockSpec((te, d), lambda i, j: (i, 0)),
                   pl.BlockSpec((8, te), lambda i, j: (0, i))],
        out_shape=[jax.ShapeDtypeStruct((v_pad, n_e), jnp.int8),
                   jax.ShapeDtypeStruct((n_e, d), jnp.bfloat16),
                   jax.ShapeDtypeStruct((8, n_e), jnp.float32)],
        scratch_shapes=[pltpu.VMEM((d + 8, te), jnp.float32)],
    )(H, xta, ew[None, :])

    tvb = v_pad // max(1, v_pad // 512)
    nvb = v_pad // tvb
    e2tb = pl.pallas_call(
        functools.partial(_passB_body, nvb=nvb),
        grid=(nvb,),
        in_specs=[pl.BlockSpec((tvb, n_e), lambda i: (i, 0)),
                  pl.BlockSpec((tvb, d), lambda i: (i, 0)),
                  pl.BlockSpec((n_e, d), lambda i: (0, 0)),
                  pl.BlockSpec((d, d), lambda i: (0, 0)),
                  pl.BlockSpec((1, d), lambda i: (0, 0)),
                  pl.BlockSpec((8, n_e), lambda i: (0, 0)),
                  pl.BlockSpec((n_e, 128), lambda i: (0, 0))],
        out_specs=pl.BlockSpec((d, n_e), lambda i: (0, 0)),
        out_shape=jax.ShapeDtypeStruct((d, n_e), jnp.bfloat16),
        scratch_shapes=[pltpu.VMEM((d, n_e), jnp.float32)],
    )(h8, x32p, ewt16, W, bb[None, :], de8,
      jnp.ones((n_e, 128), jnp.int8))

    return e2tb


def kernel(emb_diag, g_diag, b_diag, W_diag, bb_diag, ew_diag,
           emb_proc, g_proc, b_proc, W_proc, bb_proc, ew_proc,
           emb_med, g_med, b_med, W_med, bb_med, ew_med,
           H_diag, H_proc, H_med):
    e2tb_d = _one_vocab(emb_diag, g_diag, b_diag, W_diag, bb_diag,
                        ew_diag, H_diag)
    e2tb_p = _one_vocab(emb_proc, g_proc, b_proc, W_proc, bb_proc,
                        ew_proc, H_proc)
    e2tb_m = _one_vocab(emb_med, g_med, b_med, W_med, bb_med,
                        ew_med, H_med)

    d, n_e = e2tb_d.shape
    te = 1024 if n_e > 1024 else n_e
    nte = -(-n_e // te)
    return pl.pallas_call(
        _combine_body,
        grid=(nte,),
        in_specs=[pl.BlockSpec((d, te), lambda i: (0, i)),
                  pl.BlockSpec((d, te), lambda i: (0, i)),
                  pl.BlockSpec((d, te), lambda i: (0, i))],
        out_specs=pl.BlockSpec((te, 2 * d), lambda i: (i, 0)),
        out_shape=jax.ShapeDtypeStruct((n_e, 2 * d), jnp.float32),
    )(e2tb_d, e2tb_p, e2tb_m)


# speedup vs baseline: 1.0081x; 1.0081x over previous
import functools

import jax
import jax.numpy as jnp
from jax.experimental import pallas as pl
from jax.experimental.pallas import tpu as pltpu


def _bn_body(emb_ref, g_ref, b_ref, x32_ref, xta_ref, *, v, v_pad):
    emb = emb_ref[...]
    mu = jnp.mean(emb, axis=0, keepdims=True)
    var = jnp.mean((emb - mu) ** 2, axis=0, keepdims=True)
    x = (emb - mu) * jax.lax.rsqrt(var + 1e-5) * g_ref[...] + b_ref[...]
    if v_pad > v:
        x32_ref[...] = jnp.concatenate(
            [x, jnp.zeros((v_pad - v, x.shape[1]), jnp.float32)], axis=0)
    else:
        x32_ref[...] = x
    xta = jnp.concatenate(
        [jnp.swapaxes(x.astype(jnp.bfloat16), 0, 1),
         jnp.ones((8, v), jnp.bfloat16)], axis=0)
    if v_pad > v:
        xta = jnp.concatenate(
            [xta, jnp.zeros((xta.shape[0], v_pad - v), jnp.bfloat16)], axis=1)
    xta_ref[...] = xta


def _passA_body(h_ref, xta_ref, ew_ref, h8_ref, ewt_ref, de_ref, acc_ref,
                *, v, d, tvr, nvr):
    j = pl.program_id(1)
    h = h_ref[...]
    if v % tvr:
        def _mask(hh):
            rows = jax.lax.broadcasted_iota(jnp.int32, hh.shape, 0) + j * tvr
            return jnp.where(rows < v, hh, 0.0)
        h = jax.lax.cond(j == nvr - 1, _mask, lambda hh: hh, h)
    hb = h.astype(jnp.bfloat16)
    h8_ref[...] = h.astype(jnp.int8)
    contrib = jax.lax.dot_general(xta_ref[...], hb, (((1,), (0,)), ((), ())),
                                  preferred_element_type=jnp.float32)

    @pl.when(j == 0)
    def _init():
        acc_ref[...] = contrib

    @pl.when(j > 0)
    def _acc():
        acc_ref[...] += contrib

    @pl.when(j == nvr - 1)
    def _emit():
        de = acc_ref[d:d + 8, :]
        scale = ew_ref[0:1, :] / jnp.clip(de[0:1, :], 1.0, None)
        ewt_ref[...] = jnp.swapaxes(
            (acc_ref[0:d, :] * scale).astype(jnp.bfloat16), 0, 1)
        de_ref[...] = de


def _passB_body(h8_ref, x32_ref, ewt_ref, w_ref, bb_ref, de_ref,
                ones8_ref, e2tb_ref, acc_ref, *, nvb):
    i = pl.program_id(0)
    h8 = h8_ref[...]
    hb = h8.astype(jnp.bfloat16)
    dv32 = jax.lax.dot_general(h8, ones8_ref[...], (((1,), (0,)), ((), ())),
                               preferred_element_type=jnp.int32)
    dv = dv32[:, 0:1].astype(jnp.float32)
    m = jax.lax.dot_general(hb, ewt_ref[...], (((1,), (0,)), ((), ())),
                            preferred_element_type=jnp.float32)
    m = m / jnp.clip(dv, 1.0, None)
    r = jax.nn.relu(
        jax.lax.dot_general(m.astype(jnp.bfloat16),
                            w_ref[...].astype(jnp.bfloat16),
                            (((1,), (0,)), ((), ())),
                            preferred_element_type=jnp.float32) + bb_ref[...])
    xo16 = (r + x32_ref[...]).astype(jnp.bfloat16)
    xot = jnp.swapaxes(xo16, 0, 1)
    contrib = jax.lax.dot_general(xot, hb, (((1,), (0,)), ((), ())),
                                  preferred_element_type=jnp.float32)

    @pl.when(i == 0)
    def _init():
        acc_ref[...] = contrib

    @pl.when(i > 0)
    def _acc():
        acc_ref[...] += contrib

    @pl.when(i == nvb - 1)
    def _emit():
        invde = 1.0 / jnp.clip(de_ref[0:1, :], 1.0, None)
        e2tb_ref[...] = (acc_ref[...] * invde).astype(jnp.bfloat16)


def _combine_body(ed_ref, ep_ref, em_ref, out_ref):
    dp = (ed_ref[...].astype(jnp.float32)
          + ep_ref[...].astype(jnp.float32))
    mm = em_ref[...].astype(jnp.float32)
    out_ref[...] = jnp.concatenate(
        [jnp.swapaxes(dp, 0, 1), jnp.swapaxes(mm, 0, 1)], axis=1)


def _one_vocab(emb, g, b, W, bb, ew, H):
    v, d = emb.shape
    n_e = H.shape[1]
    if v >= 512:
        tvr = 1024 if -(-v // 1024) * 1024 == -(-v // 512) * 512 else 512
    else:
        tvr = 128
    nvr = -(-v // tvr)
    v_pad = nvr * tvr
    te = 2048 if n_e > 2048 else n_e
    nte = -(-n_e // te)

    x32p, xta = pl.pallas_call(
        functools.partial(_bn_body, v=v, v_pad=v_pad),
        out_shape=[jax.ShapeDtypeStruct((v_pad, d), jnp.float32),
                   jax.ShapeDtypeStruct((d + 8, v_pad), jnp.bfloat16)],
    )(emb, g, b)

    h8, ewt16, de8 = pl.pallas_call(
        functools.partial(_passA_body, v=v, d=d, tvr=tvr, nvr=nvr),
        grid=(nte, nvr),
        in_specs=[pl.BlockSpec((tvr, te), lambda i, j: (j, i)),
                  pl.BlockSpec((d + 8, tvr), lambda i, j: (0, j)),
                  pl.BlockSpec((1, te), lambda i, j: (0, i))],
        out_specs=[pl.BlockSpec((tvr, te), lambda i, j: (j, i)),
                   pl.BlockSpec((te, d), lambda i, j: (i, 0)),
                   pl.BlockSpec((8, te), lambda i, j: (0, i))],
        out_shape=[jax.ShapeDtypeStruct((v_pad, n_e), jnp.int8),
                   jax.ShapeDtypeStruct((n_e, d), jnp.bfloat16),
                   jax.ShapeDtypeStruct((8, n_e), jnp.float32)],
        scratch_shapes=[pltpu.VMEM((d + 8, te), jnp.float32)],
    )(H, xta, ew[None, :])

    tvb = v_pad // max(1, v_pad // 512)
    nvb = v_pad // tvb
    e2tb = pl.pallas_call(
        functools.partial(_passB_body, nvb=nvb),
        grid=(nvb,),
        in_specs=[pl.BlockSpec((tvb, n_e), lambda i: (i, 0)),
                  pl.BlockSpec((tvb, d), lambda i: (i, 0)),
                  pl.BlockSpec((n_e, d), lambda i: (0, 0)),
                  pl.BlockSpec((d, d), lambda i: (0, 0)),
                  pl.BlockSpec((1, d), lambda i: (0, 0)),
                  pl.BlockSpec((8, n_e), lambda i: (0, 0)),
                  pl.BlockSpec((n_e, 128), lambda i: (0, 0))],
        out_specs=pl.BlockSpec((d, n_e), lambda i: (0, 0)),
        out_shape=jax.ShapeDtypeStruct((d, n_e), jnp.bfloat16),
        scratch_shapes=[pltpu.VMEM((d, n_e), jnp.float32)],
    )(h8, x32p, ewt16, W, bb[None, :], de8,
      jnp.ones((n_e, 128), jnp.int8))

    return e2tb


def kernel(emb_diag, g_diag, b_diag, W_diag, bb_diag, ew_diag,
           emb_proc, g_proc, b_proc, W_proc, bb_proc, ew_proc,
           emb_med, g_med, b_med, W_med, bb_med, ew_med,
           H_diag, H_proc, H_med):
    e2tb_d = _one_vocab(emb_diag, g_diag, b_diag, W_diag, bb_diag,
                        ew_diag, H_diag)
    e2tb_p = _one_vocab(emb_proc, g_proc, b_proc, W_proc, bb_proc,
                        ew_proc, H_proc)
    e2tb_m = _one_vocab(emb_med, g_med, b_med, W_med, bb_med,
                        ew_med, H_med)

    d, n_e = e2tb_d.shape
    te = 1024 if n_e > 1024 else n_e
    nte = -(-n_e // te)
    return pl.pallas_call(
        _combine_body,
        grid=(nte,),
        in_specs=[pl.BlockSpec((d, te), lambda i: (0, i)),
                  pl.BlockSpec((d, te), lambda i: (0, i)),
                  pl.BlockSpec((d, te), lambda i: (0, i))],
        out_specs=pl.BlockSpec((te, 2 * d), lambda i: (i, 0)),
        out_shape=jax.ShapeDtypeStruct((n_e, 2 * d), jnp.float32),
    )(e2tb_d, e2tb_p, e2tb_m)
